# baseline (speedup 1.0000x reference)
import functools

import jax
import jax.numpy as jnp
from jax import lax
from jax.experimental import pallas as pl
from jax.experimental.pallas import tpu as pltpu
from jax.experimental.pallas import tpu_sc as plsc

NC = 2
NS = 16
NW = NC * NS
CHUNK = 128
LANES = 16


def _spmm_sc(x, col3, row3, val3, n_chunks, n_nodes, d):
    rows_per_tile = n_nodes // NS
    n_full = rows_per_tile // CHUNK
    rem = rows_per_tile % CHUNK
    mesh = plsc.VectorSubcoreMesh(core_axis_name="c", subcore_axis_name="s")

    @functools.partial(
        pl.kernel,
        mesh=mesh,
        out_type=jax.ShapeDtypeStruct((NC, n_nodes, d), jnp.float32),
        scratch_types=[
            pltpu.VMEM((4, CHUNK), jnp.int32),
            pltpu.VMEM((4, CHUNK), jnp.int32),
            pltpu.VMEM((4, CHUNK), jnp.float32),
            pltpu.VMEM((2, CHUNK, d), jnp.float32),
            pltpu.VMEM_SHARED((n_nodes, d), jnp.float32),
            pltpu.SemaphoreType.DMA,
            pltpu.SemaphoreType.DMA,
            pltpu.SemaphoreType.DMA,
        ],
    )
    def spmm(x_hbm, col_hbm, row_hbm, val_hbm, out_hbm,
             col4, row4, val4, rows2, acc, sem_g, sem_i, sem_s):
        cid = lax.axis_index("c")
        sid = lax.axis_index("s")
        wid = sid * NC + cid

        zbuf = rows2.at[0]

        def zero_body(e, _):
            for s in range(d // LANES):
                zbuf[e, pl.ds(s * LANES, LANES)] = jnp.zeros(
                    (LANES,), jnp.float32)
            return 0
        lax.fori_loop(0, CHUNK, zero_body, 0)

        base = sid * rows_per_tile
        for b in range(n_full):
            pltpu.sync_copy(zbuf, acc.at[pl.ds(base + b * CHUNK, CHUNK)])
        if rem:
            pltpu.sync_copy(zbuf.at[pl.ds(0, rem)],
                            acc.at[pl.ds(base + n_full * CHUNK, rem)])
        plsc.subcore_barrier()

        def idx_copies(chunk, slot):
            return (
                pltpu.make_async_copy(col_hbm.at[wid, chunk], col4.at[slot],
                                      sem_i),
                pltpu.make_async_copy(row_hbm.at[wid, chunk], row4.at[slot],
                                      sem_i),
                pltpu.make_async_copy(val_hbm.at[wid, chunk], val4.at[slot],
                                      sem_i),
            )

        pltpu.sync_copy(col_hbm.at[wid, 0], col4.at[0])
        pltpu.sync_copy(row_hbm.at[wid, 0], row4.at[0])
        pltpu.sync_copy(val_hbm.at[wid, 0], val4.at[0])
        for cp in idx_copies(1, 1):
            cp.start()
        pltpu.make_async_copy(x_hbm.at[col4.at[0]], rows2.at[0],
                              sem_g).start()

        def pair_body(i, _):
            for b in range(2):
                c = i * 2 + b
                nb = 1 - b
                r = lax.rem(c, 4)
                c1 = jnp.where(c + 1 < n_chunks, c + 1, 0)
                r1 = lax.rem(c + 1, 4)
                c2 = jnp.where(c + 2 < n_chunks, c + 2, 0)
                r2 = lax.rem(c + 2, 4)
                buf = rows2.at[b]

                pltpu.make_async_copy(x_hbm.at[col4.at[r]], buf,
                                      sem_g).wait()
                for cp in idx_copies(c1, r1):
                    cp.wait()

                @pl.when(c > 0)
                def _():
                    pltpu.make_async_copy(
                        rows2.at[nb], acc.at[row4.at[lax.rem(c + 3, 4)]],
                        sem_s).wait()

                pltpu.make_async_copy(x_hbm.at[col4.at[r1]], rows2.at[nb],
                                      sem_g).start()
                for cp in idx_copies(c2, r2):
                    cp.start()

                def scale_body(g, _):
                    vg = val4[r, pl.ds(g * LANES, LANES)]
                    for j in range(LANES):
                        e = g * LANES + j
                        v = vg[j]
                        for s in range(d // LANES):
                            sl = pl.ds(s * LANES, LANES)
                            buf[e, sl] = buf[e, sl] * v
                    return 0
                lax.fori_loop(0, CHUNK // LANES, scale_body, 0)

                pltpu.async_copy(buf, acc.at[row4.at[r]], sem_s, add=True)
            return 0
        lax.fori_loop(0, n_chunks // 2, pair_body, 0)

        pltpu.make_async_copy(
            rows2.at[(n_chunks - 1) % 2],
            acc.at[row4.at[(n_chunks - 1) % 4]], sem_s).wait()
        pltpu.make_async_copy(x_hbm.at[col4.at[0]],
                              rows2.at[n_chunks % 2], sem_g).wait()
        for cp in idx_copies(0, (n_chunks + 1) % 4):
            cp.wait()
        plsc.subcore_barrier()

        pltpu.sync_copy(acc.at[pl.ds(base, rows_per_tile)],
                        out_hbm.at[cid, pl.ds(base, rows_per_tile)])

    return spmm(x, col3, row3, val3)


def _finish_tc(partial, W, n_nodes, d):
    blk = 1024

    def body(p_ref, w_ref, o_ref):
        acc = p_ref[0] + p_ref[1]
        o_ref[...] = jnp.dot(acc, w_ref[...],
                             preferred_element_type=jnp.float32)

    return pl.pallas_call(
        body,
        grid=(n_nodes // blk,),
        in_specs=[
            pl.BlockSpec((2, blk, d), lambda i: (0, i, 0)),
            pl.BlockSpec((d, d), lambda i: (0, 0)),
        ],
        out_specs=pl.BlockSpec((blk, d), lambda i: (i, 0)),
        out_shape=jax.ShapeDtypeStruct((n_nodes, d), jnp.float32),
    )(partial, W)


def kernel(x, edge_index, edge_vals, W):
    n_nodes, d = x.shape
    n_pad = -(-n_nodes // (NS * CHUNK)) * (NS * CHUNK)
    row = edge_index[0].astype(jnp.int32)
    col = edge_index[1].astype(jnp.int32)
    vals = edge_vals.astype(jnp.float32)

    e = row.shape[0]
    per_tile = -(-e // NW)
    n_chunks = -(-per_tile // CHUNK)
    n_chunks += n_chunks % 2
    e_pad = n_chunks * CHUNK * NW
    pad = e_pad - e
    row = jnp.pad(row, (0, pad)).reshape(NW, n_chunks, CHUNK)
    col = jnp.pad(col, (0, pad)).reshape(NW, n_chunks, CHUNK)
    vals = jnp.pad(vals, (0, pad)).reshape(NW, n_chunks, CHUNK)

    partial = _spmm_sc(x, col, row, vals, n_chunks, n_pad, d)
    return _finish_tc(partial, W, n_pad, d)[:n_nodes]
